# baseline (speedup 1.0000x reference)
import functools

import jax
import jax.numpy as jnp
from jax import lax
from jax.experimental import pallas as pl
from jax.experimental.pallas import tpu as pltpu
from jax.experimental.pallas import tpu_sc as plsc

B = 16384
D = 10
L = 16
NS = 16
ROWS_PER_SC = B // NS
NGROUP = ROWS_PER_SC // L


def _rowsum_body(o_ref, t_ref, rs_ref):
    d = o_ref[...] - t_ref[...]
    rs_ref[...] = jnp.sum(d * d, axis=0)


_TC_BLK = 2048

_tc_rowsums = pl.pallas_call(
    _rowsum_body,
    grid=(B // _TC_BLK,),
    in_specs=[
        pl.BlockSpec((D, _TC_BLK), lambda i: (0, i)),
        pl.BlockSpec((D, _TC_BLK), lambda i: (0, i)),
    ],
    out_specs=pl.BlockSpec((_TC_BLK,), lambda i: (i,)),
    out_shape=jax.ShapeDtypeStruct((B,), jnp.float32),
)


def _sc_body(rs_hbm, l_hbm, out_hbm,
             rs_v, lbl_v, part_v, loc_v, res_v, shared, sem):
    sid = lax.axis_index("s")
    row0 = sid * ROWS_PER_SC

    cps = [
        pltpu.async_copy(rs_hbm.at[pl.ds(row0, ROWS_PER_SC)], rs_v, sem),
        pltpu.async_copy(l_hbm.at[pl.ds(row0, ROWS_PER_SC)], lbl_v, sem),
    ]
    for cp in cps:
        cp.wait()

    ones = jnp.full((L,), 1.0, jnp.float32)
    zeros = jnp.full((L,), 0.0, jnp.float32)

    def step(g, carry):
        s_acc, c_acc = carry
        base = g * L
        lbl = lbl_v[pl.ds(base, L)]
        vf = jnp.where(lbl == -2, ones, zeros)
        rs = rs_v[pl.ds(base, L)]
        return (s_acc + rs * vf, c_acc + vf)

    s_acc, c_acc = lax.fori_loop(0, NGROUP, step, (zeros, zeros))

    part_v[pl.ds(0, L)] = s_acc
    part_v[pl.ds(L, L)] = c_acc
    pltpu.sync_copy(part_v, shared.at[pl.ds(sid * 2 * L, 2 * L)])
    plsc.subcore_barrier()

    @pl.when(sid == 0)
    def _():
        pltpu.sync_copy(shared, loc_v)

        def cstep(i, carry):
            s, c = carry
            return (s + loc_v[pl.ds(i * 2 * L, L)],
                    c + loc_v[pl.ds(i * 2 * L + L, L)])

        s_tot, c_tot = lax.fori_loop(0, NS, cstep, (zeros, zeros))
        ts = jnp.sum(s_tot)
        tc = jnp.sum(c_tot)
        res_v[...] = jnp.full((L,), ts, jnp.float32) / jnp.full(
            (L,), tc, jnp.float32)
        pltpu.sync_copy(res_v, out_hbm)


_sc_call = functools.partial(
    pl.kernel,
    mesh=plsc.VectorSubcoreMesh(core_axis_name="c", subcore_axis_name="s",
                                num_cores=1),
    out_type=jax.ShapeDtypeStruct((L,), jnp.float32),
    compiler_params=pltpu.CompilerParams(needs_layout_passes=False),
    scratch_types=[
        pltpu.VMEM((ROWS_PER_SC,), jnp.float32),
        pltpu.VMEM((ROWS_PER_SC,), jnp.int32),
        pltpu.VMEM((2 * L,), jnp.float32),
        pltpu.VMEM((NS * 2 * L,), jnp.float32),
        pltpu.VMEM((L,), jnp.float32),
        pltpu.VMEM_SHARED((NS * 2 * L,), jnp.float32),
        pltpu.SemaphoreType.DMA,
    ],
)(_sc_body)


@jax.jit
def kernel(landmark_out, landmark_target, label):
    rs = _tc_rowsums(landmark_out.T, landmark_target.T)
    out = _sc_call(rs, label.reshape(-1))
    return out[0]


# speedup vs baseline: 1.1440x; 1.1440x over previous
import functools

import jax
import jax.numpy as jnp
from jax import lax
from jax.experimental import pallas as pl
from jax.experimental.pallas import tpu as pltpu
from jax.experimental.pallas import tpu_sc as plsc

B = 16384
D = 10
L = 16
NS = 16
ROWS_PER_SC = B // NS
NGROUP = ROWS_PER_SC // L


def _rowsum_body(o_ref, t_ref, rs_ref):
    d = o_ref[...] - t_ref[...]
    rs_ref[...] = jnp.sum(d * d, axis=0)


_tc_rowsums = pl.pallas_call(
    _rowsum_body,
    out_shape=jax.ShapeDtypeStruct((B,), jnp.float32),
)


def _sc_body(rs_hbm, l_hbm, out_hbm,
             rs_v, lbl_v, part_v, loc_v, res_v, shared, sem):
    sid = lax.axis_index("s")
    row0 = sid * ROWS_PER_SC

    cps = [
        pltpu.async_copy(rs_hbm.at[pl.ds(row0, ROWS_PER_SC)], rs_v, sem),
        pltpu.async_copy(l_hbm.at[pl.ds(row0, ROWS_PER_SC)], lbl_v, sem),
    ]
    for cp in cps:
        cp.wait()

    ones = jnp.full((L,), 1.0, jnp.float32)
    zeros = jnp.full((L,), 0.0, jnp.float32)

    def step(g, carry):
        s_acc, c_acc = carry
        base = g * L
        lbl = lbl_v[pl.ds(base, L)]
        vf = jnp.where(lbl == -2, ones, zeros)
        rs = rs_v[pl.ds(base, L)]
        return (s_acc + rs * vf, c_acc + vf)

    s_acc, c_acc = lax.fori_loop(0, NGROUP, step, (zeros, zeros))

    part_v[pl.ds(0, L)] = s_acc
    part_v[pl.ds(L, L)] = c_acc
    pltpu.sync_copy(part_v, shared.at[pl.ds(sid * 2 * L, 2 * L)])
    plsc.subcore_barrier()

    @pl.when(sid == 0)
    def _():
        pltpu.sync_copy(shared, loc_v)

        def cstep(i, carry):
            s, c = carry
            return (s + loc_v[pl.ds(i * 2 * L, L)],
                    c + loc_v[pl.ds(i * 2 * L + L, L)])

        s_tot, c_tot = lax.fori_loop(0, NS, cstep, (zeros, zeros))
        ts = jnp.sum(s_tot)
        tc = jnp.sum(c_tot)
        res_v[...] = jnp.full((L,), ts, jnp.float32) / jnp.full(
            (L,), tc, jnp.float32)
        pltpu.sync_copy(res_v, out_hbm)


_sc_call = functools.partial(
    pl.kernel,
    mesh=plsc.VectorSubcoreMesh(core_axis_name="c", subcore_axis_name="s",
                                num_cores=1),
    out_type=jax.ShapeDtypeStruct((L,), jnp.float32),
    compiler_params=pltpu.CompilerParams(needs_layout_passes=False),
    scratch_types=[
        pltpu.VMEM((ROWS_PER_SC,), jnp.float32),
        pltpu.VMEM((ROWS_PER_SC,), jnp.int32),
        pltpu.VMEM((2 * L,), jnp.float32),
        pltpu.VMEM((NS * 2 * L,), jnp.float32),
        pltpu.VMEM((L,), jnp.float32),
        pltpu.VMEM_SHARED((NS * 2 * L,), jnp.float32),
        pltpu.SemaphoreType.DMA,
    ],
)(_sc_body)


@jax.jit
def kernel(landmark_out, landmark_target, label):
    rs = _tc_rowsums(landmark_out.T, landmark_target.T)
    out = _sc_call(rs, label.reshape(-1))
    return out[0]
